# baseline (speedup 1.0000x reference)
import functools

import jax
import jax.numpy as jnp
from jax.experimental import pallas as pl
from jax.experimental.pallas import tpu as pltpu

N_TOK = 16384
D_EMB = 2048
N_EXPERTS = 16
TOP_K = 2
BLK = 1024
N_BUF = 2
SPLITS = 4
SUB = BLK // SPLITS


def _h_copy(h_hbm, hbuf, sems, chunk, slot, s):
    return pltpu.make_async_copy(
        h_hbm.at[pl.ds(chunk * BLK + s * SUB, SUB), :],
        hbuf.at[slot, pl.ds(s * SUB, SUB), :],
        sems.at[slot, s])


def _router_kernel(h_hbm, md_ref, w1_ref, b1_ref, w2_ref, b2_ref,
                   wg_ref, bg_ref, gw_ref, idx_ref, hbuf, sems):
    i = pl.program_id(0)
    nsteps = pl.num_programs(0)
    slot = jax.lax.rem(i, N_BUF)

    @pl.when(i == 0)
    def _prologue():
        for s in range(SPLITS):
            _h_copy(h_hbm, hbuf, sems, i, slot, s).start()

    @pl.when(i + 1 < nsteps)
    def _prefetch():
        nslot = jax.lax.rem(i + 1, N_BUF)
        for s in range(SPLITS):
            _h_copy(h_hbm, hbuf, sems, i + 1, nslot, s).start()

    for s in range(SPLITS):
        _h_copy(h_hbm, hbuf, sems, i, slot, s).wait()
    hb = hbuf[slot]

    md = md_ref[...]
    g = jnp.dot(md, w1_ref[...], preferred_element_type=jnp.float32)
    g = g + b1_ref[...]
    g = 0.5 * g * (1.0 + jax.lax.erf(g * 0.7071067811865476))
    m_emb = jnp.dot(g, w2_ref[...], preferred_element_type=jnp.float32)
    m_emb = m_emb + b2_ref[...]

    logits = jnp.dot(hb, wg_ref[:D_EMB, :], preferred_element_type=jnp.float32)
    logits = logits + jnp.dot(m_emb, wg_ref[D_EMB:, :],
                              preferred_element_type=jnp.float32)
    logits = logits + bg_ref[...]

    cols = jax.lax.broadcasted_iota(jnp.int32, (BLK, N_EXPERTS), 1)
    idx1 = jnp.argmax(logits, axis=-1).astype(jnp.int32)
    v1 = jnp.max(logits, axis=-1)
    masked = jnp.where(cols == idx1[:, None], -jnp.inf, logits)
    idx2 = jnp.argmax(masked, axis=-1).astype(jnp.int32)
    v2 = jnp.max(masked, axis=-1)

    e = jnp.exp(v2 - v1)
    w2 = e / (1.0 + e)
    w1 = 1.0 - w2

    gw = jnp.where(cols == idx1[:, None], w1[:, None], 0.0)
    gw = jnp.where(cols == idx2[:, None], w2[:, None], gw)
    gw_ref[...] = gw
    idx_ref[...] = jnp.concatenate([idx1[:, None], idx2[:, None]], axis=-1)


@functools.partial(jax.jit, static_argnames=())
def kernel(h, metadata, W1, b1, W2, b2, Wg, bg, mu):
    n_tok = h.shape[0]
    grid = (n_tok // BLK,)
    full = lambda shape: pl.BlockSpec(shape, lambda i: (0,) * len(shape))

    gw, idx = pl.pallas_call(
        _router_kernel,
        grid=grid,
        in_specs=[
            pl.BlockSpec(memory_space=pltpu.MemorySpace.HBM),
            pl.BlockSpec((BLK, 2), lambda i: (i, 0)),
            full((2, 16)),
            full((1, 16)),
            full((16, 8)),
            full((1, 8)),
            full((D_EMB + 8, N_EXPERTS)),
            full((1, N_EXPERTS)),
        ],
        out_specs=[
            pl.BlockSpec((BLK, N_EXPERTS), lambda i: (i, 0)),
            pl.BlockSpec((BLK, TOP_K), lambda i: (i, 0)),
        ],
        out_shape=[
            jax.ShapeDtypeStruct((n_tok, N_EXPERTS), jnp.float32),
            jax.ShapeDtypeStruct((n_tok, TOP_K), jnp.int32),
        ],
        scratch_shapes=[
            pltpu.VMEM((N_BUF, BLK, D_EMB), jnp.float32),
            pltpu.SemaphoreType.DMA((N_BUF, SPLITS)),
        ],
        compiler_params=pltpu.CompilerParams(
            dimension_semantics=("arbitrary",),
        ),
    )(h, metadata, W1, b1.reshape(1, -1), W2, b2.reshape(1, -1),
      Wg, bg.reshape(1, -1))
    return (gw, idx, mu)


# speedup vs baseline: 1.0437x; 1.0437x over previous
import functools

import jax
import jax.numpy as jnp
from jax.experimental import pallas as pl
from jax.experimental.pallas import tpu as pltpu

N_TOK = 16384
D_EMB = 2048
N_EXPERTS = 16
TOP_K = 2
BLK = 512
N_BUF = 4
SPLITS = 2
SUB = BLK // SPLITS
NSTEPS = N_TOK // BLK


def _h_copy(h_hbm, hbuf, sems, chunk, slot, s):
    return pltpu.make_async_copy(
        h_hbm.at[pl.ds(chunk * BLK + s * SUB, SUB), :],
        hbuf.at[slot, pl.ds(s * SUB, SUB), :],
        sems.at[slot, s])


def _router_kernel(h_hbm, md_ref, w1_ref, b1_ref, w2_ref, b2_ref,
                   wg_ref, bg_ref, gw_ref, idx_ref, hbuf, sems):
    i = pl.program_id(0)
    slot = jax.lax.rem(i, N_BUF)

    @pl.when(i == 0)
    def _prologue():
        for c in range(min(N_BUF - 1, NSTEPS)):
            for s in range(SPLITS):
                _h_copy(h_hbm, hbuf, sems, c, c % N_BUF, s).start()

    @pl.when(i + N_BUF - 1 < NSTEPS)
    def _prefetch():
        nxt = i + N_BUF - 1
        nslot = jax.lax.rem(nxt, N_BUF)
        for s in range(SPLITS):
            _h_copy(h_hbm, hbuf, sems, nxt, nslot, s).start()

    for s in range(SPLITS):
        _h_copy(h_hbm, hbuf, sems, i, slot, s).wait()
    hb = hbuf[slot]

    md = md_ref[...]
    g = jnp.dot(md, w1_ref[...], preferred_element_type=jnp.float32)
    g = g + b1_ref[...]
    g = 0.5 * g * (1.0 + jax.lax.erf(g * 0.7071067811865476))
    m_emb = jnp.dot(g, w2_ref[...], preferred_element_type=jnp.float32)
    m_emb = m_emb + b2_ref[...]

    logits = jnp.dot(hb, wg_ref[:D_EMB, :], preferred_element_type=jnp.float32)
    logits = logits + jnp.dot(m_emb, wg_ref[D_EMB:, :],
                              preferred_element_type=jnp.float32)
    logits = logits + bg_ref[...]

    cols = jax.lax.broadcasted_iota(jnp.int32, (BLK, N_EXPERTS), 1)
    idx1 = jnp.argmax(logits, axis=-1).astype(jnp.int32)
    v1 = jnp.max(logits, axis=-1)
    masked = jnp.where(cols == idx1[:, None], -jnp.inf, logits)
    idx2 = jnp.argmax(masked, axis=-1).astype(jnp.int32)
    v2 = jnp.max(masked, axis=-1)

    e = jnp.exp(v2 - v1)
    w2 = e / (1.0 + e)
    w1 = 1.0 - w2

    gw = jnp.where(cols == idx1[:, None], w1[:, None], 0.0)
    gw = jnp.where(cols == idx2[:, None], w2[:, None], gw)
    gw_ref[...] = gw
    idx_ref[...] = jnp.concatenate([idx1[:, None], idx2[:, None]], axis=-1)


@functools.partial(jax.jit, static_argnames=())
def kernel(h, metadata, W1, b1, W2, b2, Wg, bg, mu):
    n_tok = h.shape[0]
    grid = (n_tok // BLK,)
    full = lambda shape: pl.BlockSpec(shape, lambda i: (0,) * len(shape))

    gw, idx = pl.pallas_call(
        _router_kernel,
        grid=grid,
        in_specs=[
            pl.BlockSpec(memory_space=pltpu.MemorySpace.HBM),
            pl.BlockSpec((BLK, 2), lambda i: (i, 0)),
            full((2, 16)),
            full((1, 16)),
            full((16, 8)),
            full((1, 8)),
            full((D_EMB + 8, N_EXPERTS)),
            full((1, N_EXPERTS)),
        ],
        out_specs=[
            pl.BlockSpec((BLK, N_EXPERTS), lambda i: (i, 0)),
            pl.BlockSpec((BLK, TOP_K), lambda i: (i, 0)),
        ],
        out_shape=[
            jax.ShapeDtypeStruct((n_tok, N_EXPERTS), jnp.float32),
            jax.ShapeDtypeStruct((n_tok, TOP_K), jnp.int32),
        ],
        scratch_shapes=[
            pltpu.VMEM((N_BUF, BLK, D_EMB), jnp.float32),
            pltpu.SemaphoreType.DMA((N_BUF, SPLITS)),
        ],
        compiler_params=pltpu.CompilerParams(
            dimension_semantics=("arbitrary",),
        ),
    )(h, metadata, W1, b1.reshape(1, -1), W2, b2.reshape(1, -1),
      Wg, bg.reshape(1, -1))
    return (gw, idx, mu)
